# Initial kernel scaffold; baseline (speedup 1.0000x reference)
import jax
import jax.numpy as jnp
from jax.experimental import pallas as pl


def kernel(x, edge_index, edge_attr, node_batch, params):
    raise NotImplementedError("write your pallas kernel here")


# speedup vs baseline: 2.9687x; 2.9687x over previous
import functools

import jax
import jax.numpy as jnp
from jax import lax
from jax.experimental import pallas as pl
from jax.experimental.pallas import tpu as pltpu
from jax.experimental.pallas import tpu_sc as plsc

_ATOM_DIMS = [119, 4, 12, 12, 10, 6, 6, 2, 2]
_BOND_DIMS = [5, 6, 2]
_N, _E, _G = 10000, 160000, 128
_LAT, _HID = 256, 512
_OHN = 256
_OHE = 128
_BN = 1000
_BE = 1000
_BEN = 2000

_MM = jnp.float32
_PREC = lax.Precision.DEFAULT

_AOFF = [0]
for _d in _ATOM_DIMS[:-1]:
    _AOFF.append(_AOFF[-1] + _d)
_BOFF = [0]
for _d in _BOND_DIMS[:-1]:
    _BOFF.append(_BOFF[-1] + _d)


def _dot(a, b):
    return jnp.dot(a.astype(jnp.bfloat16), b.astype(jnp.bfloat16),
                   preferred_element_type=jnp.float32)


def _dotsel(oh, v):
    f32, bf16 = jnp.float32, jnp.bfloat16
    p1 = v.astype(bf16)
    r = v - p1.astype(f32)
    p2 = r.astype(bf16)
    p3 = (r - p2.astype(f32)).astype(bf16)
    ohb = oh.astype(bf16)

    def d(q):
        return jnp.dot(ohb, q, preferred_element_type=f32)

    return d(p1) + d(p2) + d(p3)


def _full(shape):
    return pl.BlockSpec(shape, lambda *_: (0,) * len(shape))


def _enc_node_body(x_ref, w1, b1, w2, b2, w3, b3, o_ref):
    xx = x_ref[...]
    oh = jnp.zeros((_BN, _OHN), jnp.float32)
    iot = lax.broadcasted_iota(jnp.int32, (_BN, _OHN), 1)
    for f in range(9):
        oh = oh + (xx[:, f:f + 1] + _AOFF[f] == iot).astype(jnp.float32)
    h1 = jnp.maximum(_dot(oh, w1[...]) + b1[...], 0.0)
    h2 = jnp.maximum(_dot(h1, w2[...]) + b2[...], 0.0)
    o_ref[...] = _dot(h2, w3[...]) + b3[...]


def _enc_edge_body(a_ref, w1, b1, w2, b2, w3, b3, o_ref):
    aa = a_ref[...]
    oh = jnp.zeros((_BEN, _OHE), jnp.float32)
    iot = lax.broadcasted_iota(jnp.int32, (_BEN, _OHE), 1)
    for f in range(3):
        oh = oh + (aa[:, f:f + 1] + _BOFF[f] == iot).astype(jnp.float32)
    h1 = jnp.maximum(_dot(oh, w1[...]) + b1[...], 0.0)
    h2 = jnp.maximum(_dot(h1, w2[...]) + b2[...], 0.0)
    o_ref[...] = _dot(h2, w3[...]) + b3[...]


@functools.lru_cache(maxsize=None)
def _enc_node_call():
    return pl.pallas_call(
        _enc_node_body,
        grid=(_N // _BN,),
        in_specs=[pl.BlockSpec((_BN, 9), lambda i: (i, 0)),
                  _full((_OHN, _HID)), _full((1, _HID)),
                  _full((_HID, _HID)), _full((1, _HID)),
                  _full((_HID, _LAT)), _full((1, _LAT))],
        out_specs=pl.BlockSpec((_BN, _LAT), lambda i: (i, 0)),
        out_shape=jax.ShapeDtypeStruct((_N, _LAT), jnp.float32),
    )


@functools.lru_cache(maxsize=None)
def _enc_edge_call():
    return pl.pallas_call(
        _enc_edge_body,
        grid=(_E // _BEN,),
        in_specs=[pl.BlockSpec((_BEN, 3), lambda i: (i, 0)),
                  _full((_OHE, _HID)), _full((1, _HID)),
                  _full((_HID, _HID)), _full((1, _HID)),
                  _full((_HID, _LAT)), _full((1, _LAT))],
        out_specs=pl.BlockSpec((_BEN, _LAT), lambda i: (i, 0)),
        out_shape=jax.ShapeDtypeStruct((_E, _LAT), jnp.float32),
    )


def _starts_body(nbc_ref, nbr_ref, s0r_ref, s1r_ref, s0c_ref, s1c_ref):
    i = pl.program_id(0)
    oh = (nbc_ref[...] == lax.broadcasted_iota(jnp.int32, (_BN, _G), 1))
    oht = (nbr_ref[0] == lax.broadcasted_iota(jnp.int32, (_G, _BN), 0))
    cr = jnp.sum(oh.astype(jnp.float32), axis=0, keepdims=True)
    cc = jnp.sum(oht.astype(jnp.float32), axis=1, keepdims=True)

    @pl.when(i == 0)
    def _():
        s1r_ref[...] = jnp.zeros_like(s1r_ref)
        s1c_ref[...] = jnp.zeros_like(s1c_ref)

    s1r_ref[...] += cr
    s1c_ref[...] += cc

    @pl.when(i == pl.num_programs(0) - 1)
    def _():
        cnt_r = s1r_ref[...]
        cnt_c = s1c_ref[...]
        ir = lax.broadcasted_iota(jnp.int32, (_G, _G), 0)
        ic = lax.broadcasted_iota(jnp.int32, (_G, _G), 1)
        mr = (ir < ic).astype(jnp.float32)
        mc = (ic < ir).astype(jnp.float32)
        s0r = jnp.dot(cnt_r, mr, preferred_element_type=jnp.float32,
                      precision=lax.Precision.HIGHEST)
        s0c = jnp.dot(mc, cnt_c, preferred_element_type=jnp.float32,
                      precision=lax.Precision.HIGHEST)
        s0r_ref[...] = s0r
        s1r_ref[...] = s0r + cnt_r
        s0c_ref[...] = s0c
        s1c_ref[...] = s0c + cnt_c


@functools.lru_cache(maxsize=None)
def _starts_call():
    return pl.pallas_call(
        _starts_body,
        grid=(_N // _BN,),
        in_specs=[pl.BlockSpec((_BN, 1), lambda i: (i, 0)),
                  pl.BlockSpec((1, 1, _BN), lambda i: (i, 0, 0))],
        out_specs=[_full((1, _G)), _full((1, _G)),
                   _full((_G, 1)), _full((_G, 1))],
        out_shape=[jax.ShapeDtypeStruct((1, _G), jnp.float32),
                   jax.ShapeDtypeStruct((1, _G), jnp.float32),
                   jax.ShapeDtypeStruct((_G, 1), jnp.float32),
                   jax.ShapeDtypeStruct((_G, 1), jnp.float32)],
    )


def _edge_body(sc_ref, sr_ref, s0r, s1r, s0c, s1c, hs_ref, hd_ref, e_ref, u,
               w1, b1, w2, b2, w3, b3, o_ref, eagg_ref):
    scf = sc_ref[...].astype(jnp.float32)
    oh = ((scf >= s0r[...]) & (scf < s1r[...])).astype(jnp.float32)
    usel = _dotsel(oh, u[...])
    xx = jnp.concatenate([hs_ref[...], hd_ref[...], e_ref[...], usel], axis=1)
    h1 = jnp.maximum(_dot(xx, w1[...]) + b1[...], 0.0)
    h2 = jnp.maximum(_dot(h1, w2[...]) + b2[...], 0.0)
    enew = jnp.maximum(_dot(h2, w3[...]) + b3[...], 0.0)
    o_ref[...] = enew
    srf = sr_ref[0].astype(jnp.float32)
    oht = ((srf >= s0c[...]) & (srf < s1c[...])).astype(jnp.float32)
    contrib = _dotsel(oht, enew)

    @pl.when(pl.program_id(0) == 0)
    def _():
        eagg_ref[...] = jnp.zeros_like(eagg_ref)

    eagg_ref[...] += contrib


@functools.lru_cache(maxsize=None)
def _edge_call():
    nb = _E // _BE
    return pl.pallas_call(
        _edge_body,
        grid=(nb,),
        in_specs=[pl.BlockSpec((_BE, 1), lambda i: (i, 0)),
                  pl.BlockSpec((1, 1, _BE), lambda i: (i, 0, 0)),
                  _full((1, _G)), _full((1, _G)),
                  _full((_G, 1)), _full((_G, 1)),
                  pl.BlockSpec((_BE, _LAT), lambda i: (i, 0)),
                  pl.BlockSpec((_BE, _LAT), lambda i: (i + nb, 0)),
                  pl.BlockSpec((_BE, _LAT), lambda i: (i, 0)),
                  _full((_G, _LAT)),
                  _full((4 * _LAT, _HID)), _full((1, _HID)),
                  _full((_HID, _HID)), _full((1, _HID)),
                  _full((_HID, _LAT)), _full((1, _LAT))],
        out_specs=[pl.BlockSpec((_BE, _LAT), lambda i: (i, 0)),
                   _full((_G, _LAT))],
        out_shape=[jax.ShapeDtypeStruct((_E, _LAT), jnp.float32),
                   jax.ShapeDtypeStruct((_G, _LAT), jnp.float32)],
    )


def _node_body(nbc_ref, nbr_ref, h_ref, s_ref, r_ref, u,
               w1, b1, w2, b2, w3, b3, o_ref, nagg_ref):
    nbc = nbc_ref[...]
    oh = (nbc == lax.broadcasted_iota(jnp.int32, (_BN, _G), 1)).astype(jnp.float32)
    usel = _dotsel(oh, u[...])
    xx = jnp.concatenate([h_ref[...], s_ref[...], r_ref[...], usel], axis=1)
    h1 = jnp.maximum(_dot(xx, w1[...]) + b1[...], 0.0)
    h2 = jnp.maximum(_dot(h1, w2[...]) + b2[...], 0.0)
    hnew = jnp.maximum(_dot(h2, w3[...]) + b3[...], 0.0)
    o_ref[...] = hnew
    nbr = nbr_ref[0]
    oht = (nbr == lax.broadcasted_iota(jnp.int32, (_G, _BN), 0)).astype(jnp.float32)
    contrib = _dotsel(oht, hnew)

    @pl.when(pl.program_id(0) == 0)
    def _():
        nagg_ref[...] = jnp.zeros_like(nagg_ref)

    nagg_ref[...] += contrib


@functools.lru_cache(maxsize=None)
def _node_call():
    return pl.pallas_call(
        _node_body,
        grid=(_N // _BN,),
        in_specs=[pl.BlockSpec((_BN, 1), lambda i: (i, 0)),
                  pl.BlockSpec((1, 1, _BN), lambda i: (i, 0, 0)),
                  pl.BlockSpec((_BN, _LAT), lambda i: (i, 0)),
                  pl.BlockSpec((_BN, _LAT), lambda i: (i, 0)),
                  pl.BlockSpec((_BN, _LAT), lambda i: (i, 0)),
                  _full((_G, _LAT)),
                  _full((4 * _LAT, _HID)), _full((1, _HID)),
                  _full((_HID, _HID)), _full((1, _HID)),
                  _full((_HID, _LAT)), _full((1, _LAT))],
        out_specs=[pl.BlockSpec((_BN, _LAT), lambda i: (i, 0)),
                   _full((_G, _LAT))],
        out_shape=[jax.ShapeDtypeStruct((_N, _LAT), jnp.float32),
                   jax.ShapeDtypeStruct((_G, _LAT), jnp.float32)],
    )


def _glob_body(nagg, eagg, u, w1, b1, w2, b2, w3, b3, u_out):
    xx = jnp.concatenate([nagg[...], eagg[...], u[...]], axis=1)
    h1 = jnp.maximum(_dot(xx, w1[...]) + b1[...], 0.0)
    h2 = jnp.maximum(_dot(h1, w2[...]) + b2[...], 0.0)
    u_out[...] = jnp.maximum(_dot(h2, w3[...]) + b3[...], 0.0)


@functools.lru_cache(maxsize=None)
def _glob_call():
    return pl.pallas_call(
        _glob_body,
        grid=(1,),
        in_specs=[_full((_G, _LAT)), _full((_G, _LAT)), _full((_G, _LAT)),
                  _full((3 * _LAT, _HID)), _full((1, _HID)),
                  _full((_HID, _HID)), _full((1, _HID)),
                  _full((_HID, _LAT)), _full((1, _LAT))],
        out_specs=_full((_G, _LAT)),
        out_shape=jax.ShapeDtypeStruct((_G, _LAT), jnp.float32),
    )


def _dec_body(p_ref, w1, b1, w2, b2, w3, b3, o_ref):
    h1 = jnp.maximum(_dot(p_ref[...], w1[...]) + b1[...], 0.0)
    h2 = jnp.maximum(_dot(h1, w2[...]) + b2[...], 0.0)
    o_ref[...] = _dot(h2, w3[...]) + b3[...]


@functools.lru_cache(maxsize=None)
def _dec_call():
    return pl.pallas_call(
        _dec_body,
        grid=(1,),
        in_specs=[_full((_G, _LAT)),
                  _full((_LAT, _HID)), _full((1, _HID)),
                  _full((_HID, _HID)), _full((1, _HID)),
                  _full((_HID, 128)), _full((1, 128))],
        out_specs=_full((_G, 128)),
        out_shape=jax.ShapeDtypeStruct((_G, 128), jnp.float32),
    )


def _gather_body(x_hbm, i_hbm, o_hbm, *, nwin, d):
    def body(i_vmem, o_vmem):
        pltpu.sync_copy(x_hbm.at[i_vmem.at[0]], o_vmem)

    pltpu.emit_pipeline(
        body,
        grid=(nwin,),
        in_specs=[pl.BlockSpec((1, 128), lambda i: (0, i))],
        out_specs=[pl.BlockSpec((128, d), lambda i: (i, 0))],
        core_axis_name=("c", "s"),
        dimension_semantics=(pltpu.PARALLEL,),
    )(i_hbm, o_hbm)


@functools.lru_cache(maxsize=None)
def _gather_call(b, d, dtype):
    mesh = plsc.VectorSubcoreMesh(core_axis_name="c", subcore_axis_name="s")
    return pl.kernel(
        functools.partial(_gather_body, nwin=b // 128, d=d),
        out_type=jax.ShapeDtypeStruct((b, d), dtype),
        mesh=mesh,
    )


def _sc_gather(table, idx):
    b = idx.shape[0]
    return _gather_call(b, table.shape[1], table.dtype)(table, idx.reshape(1, b))


_NCH = _E // 128
_CPT = 80
_ROWS = 632
_ROWSL = _N - 15 * _ROWS


def _scatter_body(e_hbm, idx_hbm, z_hbm, sent_hbm, recv_hbm,
                  ibuf, vbuf, acc):
    c = lax.axis_index("c")
    s = lax.axis_index("s")
    col = c * 128
    r0 = s * _ROWS
    for t, out_hbm in ((0, sent_hbm), (1, recv_hbm)):
        @pl.when(s < 15)
        def _():
            pltpu.sync_copy(z_hbm, acc.at[pl.ds(r0, _ROWS)])

        @pl.when(s == 15)
        def _():
            pltpu.sync_copy(z_hbm.at[pl.ds(0, _ROWSL)], acc.at[pl.ds(r0, _ROWSL)])

        plsc.subcore_barrier()
        pltpu.sync_copy(idx_hbm.at[t, pl.ds(s * _CPT, _CPT)], ibuf)

        @pl.loop(0, _CPT)
        def _(k):
            ch = s * _CPT + k

            @pl.when(ch < _NCH)
            def _():
                pltpu.sync_copy(e_hbm.at[pl.ds(ch * 128, 128), pl.ds(col, 128)],
                                vbuf)
                pltpu.sync_copy(vbuf, acc.at[ibuf.at[k]], add=True)

        plsc.subcore_barrier()

        @pl.when(s < 15)
        def _():
            pltpu.sync_copy(acc.at[pl.ds(r0, _ROWS)],
                            out_hbm.at[pl.ds(r0, _ROWS), pl.ds(col, 128)])

        @pl.when(s == 15)
        def _():
            pltpu.sync_copy(acc.at[pl.ds(r0, _ROWSL)],
                            out_hbm.at[pl.ds(r0, _ROWSL), pl.ds(col, 128)])

        plsc.subcore_barrier()


@functools.lru_cache(maxsize=None)
def _scatter_call():
    mesh = plsc.VectorSubcoreMesh(core_axis_name="c", subcore_axis_name="s")
    return pl.kernel(
        _scatter_body,
        out_type=(jax.ShapeDtypeStruct((_N, _LAT), jnp.float32),
                  jax.ShapeDtypeStruct((_N, _LAT), jnp.float32)),
        mesh=mesh,
        scratch_types=[pltpu.VMEM((_CPT, 128), jnp.int32),
                       pltpu.VMEM((128, 128), jnp.float32),
                       pltpu.VMEM_SHARED((_N, 128), jnp.float32)],
    )


def _sc_scatter(e, idx2, zrows):
    return _scatter_call()(e, idx2, zrows)


def _pad_idx(idx, mult=4096):
    n = idx.shape[0]
    p = (-n) % mult
    if p:
        idx = jnp.concatenate([idx, jnp.zeros((p,), idx.dtype)])
    return idx


def kernel(x, edge_index, edge_attr, node_batch, params):
    f32 = jnp.float32
    src, dst = edge_index[0], edge_index[1]

    def wcast(a):
        return a.astype(_MM)

    def bias(b):
        return b.reshape(1, -1).astype(f32)

    (aw1, ab1), (aw2, ab2), (aw3, ab3) = params['enc_node']
    aw1p = jnp.zeros((_OHN, _HID), f32).at[:sum(_ATOM_DIMS)].set(aw1)
    h = _enc_node_call()(x, wcast(aw1p), bias(ab1), wcast(aw2), bias(ab2),
                         wcast(aw3), bias(ab3))

    (bw1, bb1), (bw2, bb2), (bw3, bb3) = params['enc_edge']
    bw1p = jnp.zeros((_OHE, _HID), f32).at[:sum(_BOND_DIMS)].set(bw1)
    e = _enc_edge_call()(edge_attr, wcast(bw1p), bias(bb1), wcast(bw2),
                         bias(bb2), wcast(bw3), bias(bb3))

    u = jnp.broadcast_to(params['global_init'], (_G, _LAT)).astype(f32)

    nbc = node_batch.reshape(_N, 1)
    nbr = node_batch.reshape(_N // _BN, 1, _BN)
    s0r, s1r, s0c, s1c = _starts_call()(nbc, nbr)
    srcc = src.reshape(_E, 1)
    srcr = src.reshape(_E // _BE, 1, _BE)
    idxp = _pad_idx(jnp.concatenate([src, dst]))
    idx2 = jnp.concatenate(
        [edge_index, jnp.zeros((2, 16 * _CPT * 128 - _E), jnp.int32)],
        axis=1).reshape(2, 16 * _CPT, 128)
    zrows = jnp.zeros((_ROWS, 128), f32)

    for lyr in params['layers']:
        (ew1, eb1), (ew2, eb2), (ew3, eb3) = lyr['edge']
        (nw1, nb1), (nw2, nb2), (nw3, nb3) = lyr['node']

        hs_hd = _sc_gather(h, idxp)
        e, eagg = _edge_call()(srcc, srcr, s0r, s1r, s0c, s1c,
                               hs_hd, hs_hd, e, u,
                               ew1, bias(eb1), ew2, bias(eb2), ew3, bias(eb3))
        sent, recv = _sc_scatter(e, idx2, zrows)
        h, nagg = _node_call()(nbc, nbr, h, sent, recv, u,
                               nw1, bias(nb1), nw2, bias(nb2), nw3, bias(nb3))
        if 'glob' in lyr:
            (gw1, gb1), (gw2, gb2), (gw3, gb3) = lyr['glob']
            u = _glob_call()(nagg, eagg, u, gw1, bias(gb1),
                             gw2, bias(gb2), gw3, bias(gb3))

    (dw1, db1), (dw2, db2), (dw3, db3) = params['decoder']
    dw3p = jnp.zeros((_HID, 128), f32).at[:, :1].set(dw3)
    db3p = jnp.zeros((128,), f32).at[:1].set(db3)
    out = _dec_call()(nagg, wcast(dw1), bias(db1), wcast(dw2), bias(db2),
                      wcast(dw3p), bias(db3p))
    return out[:, :1]
